# Initial kernel scaffold; baseline (speedup 1.0000x reference)
import jax
import jax.numpy as jnp
from jax.experimental import pallas as pl


def kernel(embedding, edge_index, sel_src_path, sel_dst_od, odNum, gat_W, attn_l, attn_r, gat_bias, mlp_W0, mlp_b0, mlp_W1, mlp_b1, mlp_W2, mlp_b2, mlp_W3, mlp_b3, mlp_W4, mlp_b4, mlp_W5, mlp_b5, mlp_W6, mlp_b6, lt_W):
    raise NotImplementedError("write your pallas kernel here")


# speedup vs baseline: 35.8624x; 35.8624x over previous
import functools

import jax
import jax.numpy as jnp
from jax import lax
from jax.experimental import pallas as pl
from jax.experimental.pallas import tpu as pltpu
from jax.experimental.pallas import tpu_sc as plsc

N = 10000
E_PP = 320000
F_IN = 128
F_OUT = 16
H = 4
FH = H * F_OUT
PK = 80
SEQ = 5
NP_ = N // SEQ
NOD = 500
ESEL = 8000

NCORES = 2
NSUB = 16
CHUNK = 128
CH_PER_TILE = 81
EW = 16
E_PAD = NCORES * NSUB * CH_PER_TILE * CHUNK
NROW = 10240
RPT = NROW // NSUB


def _p1_body(emb_ref, w_ref, al_ref, ar_ref, packed_ref, er_ref):
    feat = jnp.dot(emb_ref[...], w_ref[...], preferred_element_type=jnp.float32)
    el = jnp.dot(feat, al_ref[...], preferred_element_type=jnp.float32)
    er = jnp.dot(feat, ar_ref[...], preferred_element_type=jnp.float32)
    z = jnp.zeros((feat.shape[0], PK - FH - H), jnp.float32)
    packed_ref[...] = jnp.concatenate([feat, el, z], axis=1)
    zr = jnp.zeros((feat.shape[0], EW - H), jnp.float32)
    er_ref[...] = jnp.concatenate([er, zr], axis=1)


def _p1(embedding, gat_W, Al, Ar):
    blk = 2000
    return pl.pallas_call(
        _p1_body,
        grid=(N // blk,),
        in_specs=[
            pl.BlockSpec((blk, F_IN), lambda i: (i, 0)),
            pl.BlockSpec((F_IN, FH), lambda i: (0, 0)),
            pl.BlockSpec((FH, H), lambda i: (0, 0)),
            pl.BlockSpec((FH, H), lambda i: (0, 0)),
        ],
        out_specs=[
            pl.BlockSpec((blk, PK), lambda i: (i, 0)),
            pl.BlockSpec((blk, EW), lambda i: (i, 0)),
        ],
        out_shape=[
            jax.ShapeDtypeStruct((N, PK), jnp.float32),
            jax.ShapeDtypeStruct((N, EW), jnp.float32),
        ],
    )(embedding, gat_W, Al, Ar)


def _p2_body(packed_hbm, er_hbm, src_hbm, dst_hbm, z64_hbm, z4_hbm,
             acc_out, exs_out,
             idx_s, idx_d, G, G64, ER2, EXR, acc_sh, exs_sh, sem):
    cid = lax.axis_index("c")
    sid = lax.axis_index("s")

    pltpu.sync_copy(z64_hbm.at[pl.ds(sid * RPT, RPT), :],
                    acc_sh.at[pl.ds(sid * RPT, RPT), :])
    pltpu.sync_copy(z4_hbm.at[pl.ds(sid * RPT, RPT), :],
                    exs_sh.at[pl.ds(sid * RPT, RPT), :])

    def zr_body(r, carry):
        EXR[r, :] = jnp.zeros((EW,), jnp.float32)
        return carry

    lax.fori_loop(0, CHUNK, zr_body, 0)
    plsc.subcore_barrier()

    tile = cid * NSUB + sid
    base0 = tile * (CH_PER_TILE * CHUNK)
    lanes = lax.iota(jnp.int32, 16)

    def chunk_body(c, carry):
        base = base0 + c * CHUNK
        pltpu.sync_copy(src_hbm.at[pl.ds(base, CHUNK)], idx_s)
        pltpu.sync_copy(dst_hbm.at[pl.ds(base, CHUNK)], idx_d)
        pltpu.async_copy(packed_hbm.at[idx_s], G, sem).wait()
        pltpu.async_copy(er_hbm.at[idx_d], ER2, sem).wait()

        def grp_body(g, carry2):
            j = g * 16 + lanes
            for h in range(H):
                col = jnp.full((16,), FH + h, jnp.int32)
                elv = plsc.load_gather(G, [j, col])
                erv = plsc.load_gather(ER2, [j, jnp.full((16,), h, jnp.int32)])
                s = elv + erv
                e = jnp.where(s > 0.0, s, 0.2 * s)
                ex = jnp.exp(e)
                plsc.store_scatter(EXR, [j, jnp.full((16,), h, jnp.int32)], ex)
                for cc in range(F_OUT):
                    fc = jnp.full((16,), h * F_OUT + cc, jnp.int32)
                    v = plsc.load_gather(G, [j, fc]) * ex
                    plsc.store_scatter(G64, [j, fc], v)
            return carry2

        lax.fori_loop(0, CHUNK // 16, grp_body, 0)
        pltpu.sync_copy(G64, acc_sh.at[idx_d], add=True)
        pltpu.sync_copy(EXR, exs_sh.at[idx_d], add=True)
        return carry

    lax.fori_loop(0, CH_PER_TILE, chunk_body, 0)
    plsc.subcore_barrier()
    pltpu.sync_copy(acc_sh.at[pl.ds(sid * RPT, RPT), :],
                    acc_out.at[cid, pl.ds(sid * RPT, RPT), :])
    pltpu.sync_copy(exs_sh.at[pl.ds(sid * RPT, RPT), :],
                    exs_out.at[cid, pl.ds(sid * RPT, RPT), :])


def _p2(packed, er, src_pad, dst_pad, zeros64, zeros4):
    mesh = plsc.VectorSubcoreMesh(core_axis_name="c", subcore_axis_name="s",
                                  num_cores=NCORES)
    kern = functools.partial(
        pl.kernel,
        mesh=mesh,
        compiler_params=pltpu.CompilerParams(needs_layout_passes=False,
                                              use_tc_tiling_on_sc=False),
        out_type=[
            jax.ShapeDtypeStruct((NCORES, NROW, FH), jnp.float32),
            jax.ShapeDtypeStruct((NCORES, NROW, EW), jnp.float32),
        ],
        scratch_types=[
            pltpu.VMEM((CHUNK,), jnp.int32),
            pltpu.VMEM((CHUNK,), jnp.int32),
            pltpu.VMEM((CHUNK, PK), jnp.float32),
            pltpu.VMEM((CHUNK, FH), jnp.float32),
            pltpu.VMEM((CHUNK, EW), jnp.float32),
            pltpu.VMEM((CHUNK, EW), jnp.float32),
            pltpu.VMEM_SHARED((NROW, FH), jnp.float32),
            pltpu.VMEM_SHARED((NROW, EW), jnp.float32),
            pltpu.SemaphoreType.DMA,
        ],
    )(_p2_body)
    return kern(packed, er, src_pad, dst_pad, zeros64, zeros4)


def _p3_body(acc_ref, exs_ref, f_ref, bias_ref, rst_ref):
    a = jnp.sum(acc_ref[...], axis=0)
    s = jnp.sum(exs_ref[...], axis=0)
    d = jnp.dot(s, f_ref[...], preferred_element_type=jnp.float32)
    rst_ref[...] = a / d + bias_ref[...]


def _p3(acc2, exs2, Fexp, bias):
    blk = 2000
    return pl.pallas_call(
        _p3_body,
        grid=(N // blk,),
        in_specs=[
            pl.BlockSpec((NCORES, blk, FH), lambda i: (0, i, 0)),
            pl.BlockSpec((NCORES, blk, EW), lambda i: (0, i, 0)),
            pl.BlockSpec((EW, FH), lambda i: (0, 0)),
            pl.BlockSpec((1, FH), lambda i: (0, 0)),
        ],
        out_specs=pl.BlockSpec((blk, FH), lambda i: (i, 0)),
        out_shape=jax.ShapeDtypeStruct((N, FH), jnp.float32),
    )(acc2, exs2, Fexp, bias)


def _p4_body(gat_ref, ssc_ref, ssr_ref, sdc_ref, sdr_ref, od_ref,
             lt_ref, *wb_refs):
    mlp = wb_refs[:14]
    pf_ref, sp_ref = wb_refs[14], wb_refs[15]
    g = gat_ref[...]
    h = g
    for i in range(7):
        h = jnp.dot(h, mlp[2 * i][...], preferred_element_type=jnp.float32)
        h = h + mlp[2 * i + 1][...]
        if i < 6:
            h = jnp.maximum(h, 0.0)
    score = h + jnp.dot(g, lt_ref[...], preferred_element_type=jnp.float32)

    nch = 4
    ce = ESEL // nch
    exs = []
    s_od = jnp.zeros((NOD, 1), jnp.float32)
    for c in range(nch):
        sc = ssc_ref[pl.ds(c * ce, ce), :]
        oh_src = (jnp.broadcast_to(sc, (ce, NP_)) ==
                  lax.broadcasted_iota(jnp.int32, (ce, NP_), 1)).astype(jnp.float32)
        he = jnp.dot(oh_src, score, preferred_element_type=jnp.float32)
        ex = jnp.exp(he)
        exs.append(ex)
        dr = sdr_ref[:, pl.ds(c * ce, ce)]
        oh_odT = (lax.broadcasted_iota(jnp.int32, (NOD, ce), 0) ==
                  jnp.broadcast_to(dr, (NOD, ce))).astype(jnp.float32)
        s_od = s_od + jnp.dot(oh_odT, ex, preferred_element_type=jnp.float32)

    pf = jnp.zeros((NP_, 1), jnp.float32)
    for c in range(nch):
        dc = sdc_ref[pl.ds(c * ce, ce), :]
        oh_od = (jnp.broadcast_to(dc, (ce, NOD)) ==
                 lax.broadcasted_iota(jnp.int32, (ce, NOD), 1)).astype(jnp.float32)
        denom = jnp.dot(oh_od, s_od, preferred_element_type=jnp.float32)
        prob = exs[c] / denom
        sp_ref[pl.ds(c * ce, ce), :] = prob
        odn = jnp.dot(oh_od, od_ref[...], preferred_element_type=jnp.float32)
        sr = ssr_ref[:, pl.ds(c * ce, ce)]
        oh_srcT = (lax.broadcasted_iota(jnp.int32, (NP_, ce), 0) ==
                   jnp.broadcast_to(sr, (NP_, ce))).astype(jnp.float32)
        pf = pf + jnp.dot(oh_srcT, odn * prob, preferred_element_type=jnp.float32)
    pf_ref[...] = pf


def _p4(gatEmb, ssc, ssr, sdc, sdr, odNum, lt_W, mlp_wb):
    return pl.pallas_call(
        _p4_body,
        out_shape=[
            jax.ShapeDtypeStruct((NP_, 1), jnp.float32),
            jax.ShapeDtypeStruct((ESEL, 1), jnp.float32),
        ],
    )(gatEmb, ssc, ssr, sdc, sdr, odNum, lt_W, *mlp_wb)


@jax.jit
def kernel(embedding, edge_index, sel_src_path, sel_dst_od, odNum, gat_W,
           attn_l, attn_r, gat_bias, mlp_W0, mlp_b0, mlp_W1, mlp_b1, mlp_W2,
           mlp_b2, mlp_W3, mlp_b3, mlp_W4, mlp_b4, mlp_W5, mlp_b5, mlp_W6,
           mlp_b6, lt_W):
    f32 = jnp.float32
    hh = jnp.arange(FH, dtype=jnp.int32) // F_OUT
    sel = (hh[:, None] == jnp.arange(H, dtype=jnp.int32)[None, :]).astype(f32)
    Al = sel * attn_l.reshape(FH)[:, None]
    Ar = sel * attn_r.reshape(FH)[:, None]

    packed, er = _p1(embedding, gat_W, Al, Ar)
    er = jnp.concatenate([er, jnp.zeros((NROW - N, EW), f32)], axis=0)

    loop = jnp.arange(N, dtype=jnp.int32)
    npad = E_PAD - E_PP - N
    src_pad = jnp.concatenate([edge_index[0], loop, jnp.zeros((npad,), jnp.int32)])
    dst_pad = jnp.concatenate([edge_index[1], loop, jnp.full((npad,), N, jnp.int32)])
    zeros64 = jnp.zeros((NROW, FH), f32)
    zeros4 = jnp.zeros((NROW, EW), f32)

    acc2, exs2 = _p2(packed, er, src_pad, dst_pad, zeros64, zeros4)

    Fexp = (jnp.arange(EW, dtype=jnp.int32)[:, None] == hh[None, :]).astype(f32)
    rst = _p3(acc2, exs2, Fexp, gat_bias.reshape(1, FH))

    gatEmb = rst.reshape(NP_, FH * SEQ)
    mlp_wb = [mlp_W0, mlp_b0.reshape(1, -1), mlp_W1, mlp_b1.reshape(1, -1),
              mlp_W2, mlp_b2.reshape(1, -1), mlp_W3, mlp_b3.reshape(1, -1),
              mlp_W4, mlp_b4.reshape(1, -1), mlp_W5, mlp_b5.reshape(1, -1),
              mlp_W6, mlp_b6.reshape(1, -1)]
    predFlow, selectProb = _p4(
        gatEmb,
        sel_src_path.reshape(ESEL, 1), sel_src_path.reshape(1, ESEL),
        sel_dst_od.reshape(ESEL, 1), sel_dst_od.reshape(1, ESEL),
        odNum, lt_W, mlp_wb)
    return predFlow, selectProb


# speedup vs baseline: 43.8559x; 1.2229x over previous
import functools

import jax
import jax.numpy as jnp
from jax import lax
from jax.experimental import pallas as pl
from jax.experimental.pallas import tpu as pltpu
from jax.experimental.pallas import tpu_sc as plsc

N = 10000
E_PP = 320000
F_IN = 128
F_OUT = 16
H = 4
FH = H * F_OUT
PK = 80
SEQ = 5
NP_ = N // SEQ
NOD = 500
ESEL = 8000

NCORES = 2
NSUB = 16
CHUNK = 128
CH_PER_TILE = 81
EW = 16
E_PAD = NCORES * NSUB * CH_PER_TILE * CHUNK
NROW = 10240
RPT = NROW // NSUB


def _p1_body(emb_ref, w_ref, al_ref, ar_ref, packed_ref, er_ref):
    feat = jnp.dot(emb_ref[...], w_ref[...], preferred_element_type=jnp.float32)
    el = jnp.dot(feat, al_ref[...], preferred_element_type=jnp.float32)
    er = jnp.dot(feat, ar_ref[...], preferred_element_type=jnp.float32)
    z = jnp.zeros((feat.shape[0], PK - FH - H), jnp.float32)
    packed_ref[...] = jnp.concatenate([feat, el, z], axis=1)
    zr = jnp.zeros((feat.shape[0], EW - H), jnp.float32)
    er_ref[...] = jnp.concatenate([er, zr], axis=1)


def _p1(embedding, gat_W, Al, Ar):
    blk = 2000
    return pl.pallas_call(
        _p1_body,
        grid=(N // blk,),
        in_specs=[
            pl.BlockSpec((blk, F_IN), lambda i: (i, 0)),
            pl.BlockSpec((F_IN, FH), lambda i: (0, 0)),
            pl.BlockSpec((FH, H), lambda i: (0, 0)),
            pl.BlockSpec((FH, H), lambda i: (0, 0)),
        ],
        out_specs=[
            pl.BlockSpec((blk, PK), lambda i: (i, 0)),
            pl.BlockSpec((blk, EW), lambda i: (i, 0)),
        ],
        out_shape=[
            jax.ShapeDtypeStruct((N, PK), jnp.float32),
            jax.ShapeDtypeStruct((N, EW), jnp.float32),
        ],
    )(embedding, gat_W, Al, Ar)


def _p2_body(packed_hbm, er_hbm, src_hbm, dst_hbm, z64_hbm, z4_hbm,
             acc_out, exs_out,
             idx_sa, idx_da, idx_sb, idx_db, Ga, Gb, ERa, ERb,
             G64, EXR, acc_sh, exs_sh, sema, semb):
    cid = lax.axis_index("c")
    sid = lax.axis_index("s")

    pltpu.sync_copy(z64_hbm.at[pl.ds(sid * RPT, RPT), :],
                    acc_sh.at[pl.ds(sid * RPT, RPT), :])
    pltpu.sync_copy(z4_hbm.at[pl.ds(sid * RPT, RPT), :],
                    exs_sh.at[pl.ds(sid * RPT, RPT), :])

    def zr_body(r, carry):
        EXR[r, :] = jnp.zeros((EW,), jnp.float32)
        return carry

    lax.fori_loop(0, CHUNK, zr_body, 0)
    plsc.subcore_barrier()

    tile = cid * NSUB + sid
    base0 = tile * (CH_PER_TILE * CHUNK)
    lanes = lax.iota(jnp.int32, 16)

    def load_start(c, idx_s, idx_d, G, ER, sem):
        base = base0 + c * CHUNK
        pltpu.sync_copy(src_hbm.at[pl.ds(base, CHUNK)], idx_s)
        pltpu.sync_copy(dst_hbm.at[pl.ds(base, CHUNK)], idx_d)
        pltpu.async_copy(packed_hbm.at[idx_s], G, sem)
        pltpu.async_copy(er_hbm.at[idx_d], ER, sem)

    def wait_g(idx_s, idx_d, G, ER, sem):
        pltpu.make_async_copy(packed_hbm.at[idx_s], G, sem).wait()
        pltpu.make_async_copy(er_hbm.at[idx_d], ER, sem).wait()

    def compute_scatter(idx_d, G, ER):
        def grp_body(g, carry2):
            j = g * 16 + lanes
            for h in range(H):
                col = jnp.full((16,), FH + h, jnp.int32)
                elv = plsc.load_gather(G, [j, col])
                erv = plsc.load_gather(ER, [j, jnp.full((16,), h, jnp.int32)])
                s = elv + erv
                e = jnp.where(s > 0.0, s, 0.2 * s)
                ex = jnp.exp(e)
                plsc.store_scatter(EXR, [j, jnp.full((16,), h, jnp.int32)], ex)
                for cc in range(F_OUT):
                    fc = jnp.full((16,), h * F_OUT + cc, jnp.int32)
                    v = plsc.load_gather(G, [j, fc]) * ex
                    plsc.store_scatter(G64, [j, fc], v)
            return carry2

        lax.fori_loop(0, CHUNK // 16, grp_body, 0)
        pltpu.sync_copy(G64, acc_sh.at[idx_d], add=True)
        pltpu.sync_copy(EXR, exs_sh.at[idx_d], add=True)

    load_start(0, idx_sa, idx_da, Ga, ERa, sema)

    def pair_body(p, carry):
        c0 = 2 * p
        load_start(c0 + 1, idx_sb, idx_db, Gb, ERb, semb)
        wait_g(idx_sa, idx_da, Ga, ERa, sema)
        compute_scatter(idx_da, Ga, ERa)
        load_start(c0 + 2, idx_sa, idx_da, Ga, ERa, sema)
        wait_g(idx_sb, idx_db, Gb, ERb, semb)
        compute_scatter(idx_db, Gb, ERb)
        return carry

    lax.fori_loop(0, (CH_PER_TILE - 1) // 2, pair_body, 0)
    wait_g(idx_sa, idx_da, Ga, ERa, sema)
    compute_scatter(idx_da, Ga, ERa)
    plsc.subcore_barrier()
    pltpu.sync_copy(acc_sh.at[pl.ds(sid * RPT, RPT), :],
                    acc_out.at[cid, pl.ds(sid * RPT, RPT), :])
    pltpu.sync_copy(exs_sh.at[pl.ds(sid * RPT, RPT), :],
                    exs_out.at[cid, pl.ds(sid * RPT, RPT), :])


def _p2(packed, er, src_pad, dst_pad, zeros64, zeros4):
    mesh = plsc.VectorSubcoreMesh(core_axis_name="c", subcore_axis_name="s",
                                  num_cores=NCORES)
    kern = functools.partial(
        pl.kernel,
        mesh=mesh,
        compiler_params=pltpu.CompilerParams(needs_layout_passes=False,
                                              use_tc_tiling_on_sc=False),
        out_type=[
            jax.ShapeDtypeStruct((NCORES, NROW, FH), jnp.float32),
            jax.ShapeDtypeStruct((NCORES, NROW, EW), jnp.float32),
        ],
        scratch_types=[
            pltpu.VMEM((CHUNK,), jnp.int32),
            pltpu.VMEM((CHUNK,), jnp.int32),
            pltpu.VMEM((CHUNK,), jnp.int32),
            pltpu.VMEM((CHUNK,), jnp.int32),
            pltpu.VMEM((CHUNK, PK), jnp.float32),
            pltpu.VMEM((CHUNK, PK), jnp.float32),
            pltpu.VMEM((CHUNK, EW), jnp.float32),
            pltpu.VMEM((CHUNK, EW), jnp.float32),
            pltpu.VMEM((CHUNK, FH), jnp.float32),
            pltpu.VMEM((CHUNK, EW), jnp.float32),
            pltpu.VMEM_SHARED((NROW, FH), jnp.float32),
            pltpu.VMEM_SHARED((NROW, EW), jnp.float32),
            pltpu.SemaphoreType.DMA,
            pltpu.SemaphoreType.DMA,
        ],
    )(_p2_body)
    return kern(packed, er, src_pad, dst_pad, zeros64, zeros4)


def _p3_body(acc_ref, exs_ref, f_ref, bias_ref, rst_ref):
    a = jnp.sum(acc_ref[...], axis=0)
    s = jnp.sum(exs_ref[...], axis=0)
    d = jnp.dot(s, f_ref[...], preferred_element_type=jnp.float32)
    rst_ref[...] = a / d + bias_ref[...]


def _p3(acc2, exs2, Fexp, bias):
    blk = 2000
    return pl.pallas_call(
        _p3_body,
        grid=(N // blk,),
        in_specs=[
            pl.BlockSpec((NCORES, blk, FH), lambda i: (0, i, 0)),
            pl.BlockSpec((NCORES, blk, EW), lambda i: (0, i, 0)),
            pl.BlockSpec((EW, FH), lambda i: (0, 0)),
            pl.BlockSpec((1, FH), lambda i: (0, 0)),
        ],
        out_specs=pl.BlockSpec((blk, FH), lambda i: (i, 0)),
        out_shape=jax.ShapeDtypeStruct((N, FH), jnp.float32),
    )(acc2, exs2, Fexp, bias)


def _p4_body(gat_ref, ssc_ref, ssr_ref, sdc_ref, sdr_ref, od_ref,
             lt_ref, *wb_refs):
    mlp = wb_refs[:14]
    pf_ref, sp_ref = wb_refs[14], wb_refs[15]
    g = gat_ref[...]
    h = g
    for i in range(7):
        h = jnp.dot(h, mlp[2 * i][...], preferred_element_type=jnp.float32)
        h = h + mlp[2 * i + 1][...]
        if i < 6:
            h = jnp.maximum(h, 0.0)
    score = h + jnp.dot(g, lt_ref[...], preferred_element_type=jnp.float32)

    nch = 4
    ce = ESEL // nch
    exs = []
    s_od = jnp.zeros((NOD, 1), jnp.float32)
    for c in range(nch):
        sc = ssc_ref[pl.ds(c * ce, ce), :]
        oh_src = (jnp.broadcast_to(sc, (ce, NP_)) ==
                  lax.broadcasted_iota(jnp.int32, (ce, NP_), 1)).astype(jnp.float32)
        he = jnp.dot(oh_src, score, preferred_element_type=jnp.float32)
        ex = jnp.exp(he)
        exs.append(ex)
        dr = sdr_ref[:, pl.ds(c * ce, ce)]
        oh_odT = (lax.broadcasted_iota(jnp.int32, (NOD, ce), 0) ==
                  jnp.broadcast_to(dr, (NOD, ce))).astype(jnp.float32)
        s_od = s_od + jnp.dot(oh_odT, ex, preferred_element_type=jnp.float32)

    pf = jnp.zeros((NP_, 1), jnp.float32)
    for c in range(nch):
        dc = sdc_ref[pl.ds(c * ce, ce), :]
        oh_od = (jnp.broadcast_to(dc, (ce, NOD)) ==
                 lax.broadcasted_iota(jnp.int32, (ce, NOD), 1)).astype(jnp.float32)
        denom = jnp.dot(oh_od, s_od, preferred_element_type=jnp.float32)
        prob = exs[c] / denom
        sp_ref[pl.ds(c * ce, ce), :] = prob
        odn = jnp.dot(oh_od, od_ref[...], preferred_element_type=jnp.float32)
        sr = ssr_ref[:, pl.ds(c * ce, ce)]
        oh_srcT = (lax.broadcasted_iota(jnp.int32, (NP_, ce), 0) ==
                   jnp.broadcast_to(sr, (NP_, ce))).astype(jnp.float32)
        pf = pf + jnp.dot(oh_srcT, odn * prob, preferred_element_type=jnp.float32)
    pf_ref[...] = pf


def _p4(gatEmb, ssc, ssr, sdc, sdr, odNum, lt_W, mlp_wb):
    return pl.pallas_call(
        _p4_body,
        out_shape=[
            jax.ShapeDtypeStruct((NP_, 1), jnp.float32),
            jax.ShapeDtypeStruct((ESEL, 1), jnp.float32),
        ],
    )(gatEmb, ssc, ssr, sdc, sdr, odNum, lt_W, *mlp_wb)


@jax.jit
def kernel(embedding, edge_index, sel_src_path, sel_dst_od, odNum, gat_W,
           attn_l, attn_r, gat_bias, mlp_W0, mlp_b0, mlp_W1, mlp_b1, mlp_W2,
           mlp_b2, mlp_W3, mlp_b3, mlp_W4, mlp_b4, mlp_W5, mlp_b5, mlp_W6,
           mlp_b6, lt_W):
    f32 = jnp.float32
    hh = jnp.arange(FH, dtype=jnp.int32) // F_OUT
    sel = (hh[:, None] == jnp.arange(H, dtype=jnp.int32)[None, :]).astype(f32)
    Al = sel * attn_l.reshape(FH)[:, None]
    Ar = sel * attn_r.reshape(FH)[:, None]

    packed, er = _p1(embedding, gat_W, Al, Ar)
    er = jnp.concatenate([er, jnp.zeros((NROW - N, EW), f32)], axis=0)

    loop = jnp.arange(N, dtype=jnp.int32)
    npad = E_PAD - E_PP - N
    src_pad = jnp.concatenate([edge_index[0], loop, jnp.zeros((npad,), jnp.int32)])
    dst_pad = jnp.concatenate([edge_index[1], loop, jnp.full((npad,), N, jnp.int32)])
    zeros64 = jnp.zeros((NROW, FH), f32)
    zeros4 = jnp.zeros((NROW, EW), f32)

    acc2, exs2 = _p2(packed, er, src_pad, dst_pad, zeros64, zeros4)

    Fexp = (jnp.arange(EW, dtype=jnp.int32)[:, None] == hh[None, :]).astype(f32)
    rst = _p3(acc2, exs2, Fexp, gat_bias.reshape(1, FH))

    gatEmb = rst.reshape(NP_, FH * SEQ)
    mlp_wb = [mlp_W0, mlp_b0.reshape(1, -1), mlp_W1, mlp_b1.reshape(1, -1),
              mlp_W2, mlp_b2.reshape(1, -1), mlp_W3, mlp_b3.reshape(1, -1),
              mlp_W4, mlp_b4.reshape(1, -1), mlp_W5, mlp_b5.reshape(1, -1),
              mlp_W6, mlp_b6.reshape(1, -1)]
    predFlow, selectProb = _p4(
        gatEmb,
        sel_src_path.reshape(ESEL, 1), sel_src_path.reshape(1, ESEL),
        sel_dst_od.reshape(ESEL, 1), sel_dst_od.reshape(1, ESEL),
        odNum, lt_W, mlp_wb)
    return predFlow, selectProb


# speedup vs baseline: 46.0095x; 1.0491x over previous
import functools

import jax
import jax.numpy as jnp
from jax import lax
from jax.experimental import pallas as pl
from jax.experimental.pallas import tpu as pltpu
from jax.experimental.pallas import tpu_sc as plsc

N = 10000
E_PP = 320000
F_IN = 128
F_OUT = 16
H = 4
FH = H * F_OUT
PK = 80
SEQ = 5
NP_ = N // SEQ
NOD = 500
ESEL = 8000

NCORES = 2
NSUB = 16
CHUNK = 128
CH_PER_TILE = 81
BLKCH = 9
EW = 16
E_PAD = NCORES * NSUB * CH_PER_TILE * CHUNK
NROW = 10240
RPT = NROW // NSUB


def _p1_body(emb_ref, w_ref, al_ref, ar_ref, packed_ref, er_ref):
    feat = jnp.dot(emb_ref[...], w_ref[...], preferred_element_type=jnp.float32)
    el = jnp.dot(feat, al_ref[...], preferred_element_type=jnp.float32)
    er = jnp.dot(feat, ar_ref[...], preferred_element_type=jnp.float32)
    z = jnp.zeros((feat.shape[0], PK - FH - H), jnp.float32)
    packed_ref[...] = jnp.concatenate([feat, el, z], axis=1)
    zr = jnp.zeros((feat.shape[0], EW - H), jnp.float32)
    er_ref[...] = jnp.concatenate([er, zr], axis=1)


def _p1(embedding, gat_W, Al, Ar):
    blk = 2000
    return pl.pallas_call(
        _p1_body,
        grid=(N // blk,),
        in_specs=[
            pl.BlockSpec((blk, F_IN), lambda i: (i, 0)),
            pl.BlockSpec((F_IN, FH), lambda i: (0, 0)),
            pl.BlockSpec((FH, H), lambda i: (0, 0)),
            pl.BlockSpec((FH, H), lambda i: (0, 0)),
        ],
        out_specs=[
            pl.BlockSpec((blk, PK), lambda i: (i, 0)),
            pl.BlockSpec((blk, EW), lambda i: (i, 0)),
        ],
        out_shape=[
            jax.ShapeDtypeStruct((N, PK), jnp.float32),
            jax.ShapeDtypeStruct((N, EW), jnp.float32),
        ],
    )(embedding, gat_W, Al, Ar)


def _p2_body(packed_hbm, er_hbm, src_hbm, dst_hbm, z64_hbm, z4_hbm,
             acc_out, exs_out,
             idx_sblk, idx_dblk, Ga, Gb, ERa, ERb,
             G64, EXR, acc_sh, exs_sh, sema, semb):
    cid = lax.axis_index("c")
    sid = lax.axis_index("s")

    pltpu.sync_copy(z64_hbm.at[pl.ds(sid * RPT, RPT), :],
                    acc_sh.at[pl.ds(sid * RPT, RPT), :])
    pltpu.sync_copy(z4_hbm.at[pl.ds(sid * RPT, RPT), :],
                    exs_sh.at[pl.ds(sid * RPT, RPT), :])

    def zr_body(r, carry):
        EXR[r, :] = jnp.zeros((EW,), jnp.float32)
        return carry

    lax.fori_loop(0, CHUNK, zr_body, 0)
    plsc.subcore_barrier()

    tile = cid * NSUB + sid
    row0 = tile * CH_PER_TILE
    lanes = lax.iota(jnp.int32, 16)

    def load_start(k, G, ER, sem):
        pltpu.async_copy(packed_hbm.at[idx_sblk.at[k]], G, sem)
        pltpu.async_copy(er_hbm.at[idx_dblk.at[k]], ER, sem)

    def wait_g(k, G, ER, sem):
        pltpu.make_async_copy(packed_hbm.at[idx_sblk.at[k]], G, sem).wait()
        pltpu.make_async_copy(er_hbm.at[idx_dblk.at[k]], ER, sem).wait()

    def compute_scatter(idx_d, G, ER):
        def grp_body(g, carry2):
            j = g * 16 + lanes
            for h in range(H):
                col = jnp.full((16,), FH + h, jnp.int32)
                elv = plsc.load_gather(G, [j, col])
                erv = plsc.load_gather(ER, [j, jnp.full((16,), h, jnp.int32)])
                s = elv + erv
                e = jnp.where(s > 0.0, s, 0.2 * s)
                ex = jnp.exp(e)
                plsc.store_scatter(EXR, [j, jnp.full((16,), h, jnp.int32)], ex)
                for cc in range(F_OUT):
                    fc = jnp.full((16,), h * F_OUT + cc, jnp.int32)
                    v = plsc.load_gather(G, [j, fc]) * ex
                    plsc.store_scatter(G64, [j, fc], v)
            return carry2

        lax.fori_loop(0, CHUNK // 16, grp_body, 0)
        pltpu.sync_copy(G64, acc_sh.at[idx_d], add=True)
        pltpu.sync_copy(EXR, exs_sh.at[idx_d], add=True)

    def blk_body(b, carry):
        pltpu.sync_copy(src_hbm.at[pl.ds(row0 + b * BLKCH, BLKCH), :], idx_sblk)
        pltpu.sync_copy(dst_hbm.at[pl.ds(row0 + b * BLKCH, BLKCH), :], idx_dblk)
        load_start(0, Ga, ERa, sema)

        def pair_body(p, carry2):
            c0 = 2 * p
            load_start(c0 + 1, Gb, ERb, semb)
            wait_g(c0, Ga, ERa, sema)
            compute_scatter(idx_dblk.at[c0], Ga, ERa)
            load_start(c0 + 2, Ga, ERa, sema)
            wait_g(c0 + 1, Gb, ERb, semb)
            compute_scatter(idx_dblk.at[c0 + 1], Gb, ERb)
            return carry2

        lax.fori_loop(0, (BLKCH - 1) // 2, pair_body, 0)
        wait_g(BLKCH - 1, Ga, ERa, sema)
        compute_scatter(idx_dblk.at[BLKCH - 1], Ga, ERa)
        return carry

    lax.fori_loop(0, CH_PER_TILE // BLKCH, blk_body, 0)
    plsc.subcore_barrier()
    pltpu.sync_copy(acc_sh.at[pl.ds(sid * RPT, RPT), :],
                    acc_out.at[cid, pl.ds(sid * RPT, RPT), :])
    pltpu.sync_copy(exs_sh.at[pl.ds(sid * RPT, RPT), :],
                    exs_out.at[cid, pl.ds(sid * RPT, RPT), :])


def _p2(packed, er, src_pad, dst_pad, zeros64, zeros4):
    mesh = plsc.VectorSubcoreMesh(core_axis_name="c", subcore_axis_name="s",
                                  num_cores=NCORES)
    kern = functools.partial(
        pl.kernel,
        mesh=mesh,
        compiler_params=pltpu.CompilerParams(needs_layout_passes=False,
                                              use_tc_tiling_on_sc=False),
        out_type=[
            jax.ShapeDtypeStruct((NCORES, NROW, FH), jnp.float32),
            jax.ShapeDtypeStruct((NCORES, NROW, EW), jnp.float32),
        ],
        scratch_types=[
            pltpu.VMEM((BLKCH, CHUNK), jnp.int32),
            pltpu.VMEM((BLKCH, CHUNK), jnp.int32),
            pltpu.VMEM((CHUNK, PK), jnp.float32),
            pltpu.VMEM((CHUNK, PK), jnp.float32),
            pltpu.VMEM((CHUNK, EW), jnp.float32),
            pltpu.VMEM((CHUNK, EW), jnp.float32),
            pltpu.VMEM((CHUNK, FH), jnp.float32),
            pltpu.VMEM((CHUNK, EW), jnp.float32),
            pltpu.VMEM_SHARED((NROW, FH), jnp.float32),
            pltpu.VMEM_SHARED((NROW, EW), jnp.float32),
            pltpu.SemaphoreType.DMA,
            pltpu.SemaphoreType.DMA,
        ],
    )(_p2_body)
    return kern(packed, er, src_pad.reshape(-1, CHUNK),
                dst_pad.reshape(-1, CHUNK), zeros64, zeros4)


def _p3_body(acc_ref, exs_ref, f_ref, bias_ref, rst_ref):
    a = jnp.sum(acc_ref[...], axis=0)
    s = jnp.sum(exs_ref[...], axis=0)
    d = jnp.dot(s, f_ref[...], preferred_element_type=jnp.float32)
    rst_ref[...] = a / d + bias_ref[...]


def _p3(acc2, exs2, Fexp, bias):
    blk = 2000
    return pl.pallas_call(
        _p3_body,
        grid=(N // blk,),
        in_specs=[
            pl.BlockSpec((NCORES, blk, FH), lambda i: (0, i, 0)),
            pl.BlockSpec((NCORES, blk, EW), lambda i: (0, i, 0)),
            pl.BlockSpec((EW, FH), lambda i: (0, 0)),
            pl.BlockSpec((1, FH), lambda i: (0, 0)),
        ],
        out_specs=pl.BlockSpec((blk, FH), lambda i: (i, 0)),
        out_shape=jax.ShapeDtypeStruct((N, FH), jnp.float32),
    )(acc2, exs2, Fexp, bias)


def _p4_body(gat_ref, ssc_ref, ssr_ref, sdc_ref, sdr_ref, od_ref,
             lt_ref, *wb_refs):
    mlp = wb_refs[:14]
    pf_ref, sp_ref = wb_refs[14], wb_refs[15]
    g = gat_ref[...]
    h = g
    for i in range(7):
        h = jnp.dot(h, mlp[2 * i][...], preferred_element_type=jnp.float32)
        h = h + mlp[2 * i + 1][...]
        if i < 6:
            h = jnp.maximum(h, 0.0)
    score = h + jnp.dot(g, lt_ref[...], preferred_element_type=jnp.float32)

    nch = 4
    ce = ESEL // nch
    exs = []
    s_od = jnp.zeros((NOD, 1), jnp.float32)
    for c in range(nch):
        sc = ssc_ref[pl.ds(c * ce, ce), :]
        oh_src = (jnp.broadcast_to(sc, (ce, NP_)) ==
                  lax.broadcasted_iota(jnp.int32, (ce, NP_), 1)).astype(jnp.float32)
        he = jnp.dot(oh_src, score, preferred_element_type=jnp.float32)
        ex = jnp.exp(he)
        exs.append(ex)
        dr = sdr_ref[:, pl.ds(c * ce, ce)]
        oh_odT = (lax.broadcasted_iota(jnp.int32, (NOD, ce), 0) ==
                  jnp.broadcast_to(dr, (NOD, ce))).astype(jnp.float32)
        s_od = s_od + jnp.dot(oh_odT, ex, preferred_element_type=jnp.float32)

    pf = jnp.zeros((NP_, 1), jnp.float32)
    for c in range(nch):
        dc = sdc_ref[pl.ds(c * ce, ce), :]
        oh_od = (jnp.broadcast_to(dc, (ce, NOD)) ==
                 lax.broadcasted_iota(jnp.int32, (ce, NOD), 1)).astype(jnp.float32)
        denom = jnp.dot(oh_od, s_od, preferred_element_type=jnp.float32)
        prob = exs[c] / denom
        sp_ref[pl.ds(c * ce, ce), :] = prob
        odn = jnp.dot(oh_od, od_ref[...], preferred_element_type=jnp.float32)
        sr = ssr_ref[:, pl.ds(c * ce, ce)]
        oh_srcT = (lax.broadcasted_iota(jnp.int32, (NP_, ce), 0) ==
                   jnp.broadcast_to(sr, (NP_, ce))).astype(jnp.float32)
        pf = pf + jnp.dot(oh_srcT, odn * prob, preferred_element_type=jnp.float32)
    pf_ref[...] = pf


def _p4(gatEmb, ssc, ssr, sdc, sdr, odNum, lt_W, mlp_wb):
    return pl.pallas_call(
        _p4_body,
        out_shape=[
            jax.ShapeDtypeStruct((NP_, 1), jnp.float32),
            jax.ShapeDtypeStruct((ESEL, 1), jnp.float32),
        ],
    )(gatEmb, ssc, ssr, sdc, sdr, odNum, lt_W, *mlp_wb)


@jax.jit
def kernel(embedding, edge_index, sel_src_path, sel_dst_od, odNum, gat_W,
           attn_l, attn_r, gat_bias, mlp_W0, mlp_b0, mlp_W1, mlp_b1, mlp_W2,
           mlp_b2, mlp_W3, mlp_b3, mlp_W4, mlp_b4, mlp_W5, mlp_b5, mlp_W6,
           mlp_b6, lt_W):
    f32 = jnp.float32
    hh = jnp.arange(FH, dtype=jnp.int32) // F_OUT
    sel = (hh[:, None] == jnp.arange(H, dtype=jnp.int32)[None, :]).astype(f32)
    Al = sel * attn_l.reshape(FH)[:, None]
    Ar = sel * attn_r.reshape(FH)[:, None]

    packed, er = _p1(embedding, gat_W, Al, Ar)
    er = jnp.concatenate([er, jnp.zeros((NROW - N, EW), f32)], axis=0)

    loop = jnp.arange(N, dtype=jnp.int32)
    npad = E_PAD - E_PP - N
    src_pad = jnp.concatenate([edge_index[0], loop, jnp.zeros((npad,), jnp.int32)])
    dst_pad = jnp.concatenate([edge_index[1], loop, jnp.full((npad,), N, jnp.int32)])
    zeros64 = jnp.zeros((NROW, FH), f32)
    zeros4 = jnp.zeros((NROW, EW), f32)

    acc2, exs2 = _p2(packed, er, src_pad, dst_pad, zeros64, zeros4)

    Fexp = (jnp.arange(EW, dtype=jnp.int32)[:, None] == hh[None, :]).astype(f32)
    rst = _p3(acc2, exs2, Fexp, gat_bias.reshape(1, FH))

    gatEmb = rst.reshape(NP_, FH * SEQ)
    mlp_wb = [mlp_W0, mlp_b0.reshape(1, -1), mlp_W1, mlp_b1.reshape(1, -1),
              mlp_W2, mlp_b2.reshape(1, -1), mlp_W3, mlp_b3.reshape(1, -1),
              mlp_W4, mlp_b4.reshape(1, -1), mlp_W5, mlp_b5.reshape(1, -1),
              mlp_W6, mlp_b6.reshape(1, -1)]
    predFlow, selectProb = _p4(
        gatEmb,
        sel_src_path.reshape(ESEL, 1), sel_src_path.reshape(1, ESEL),
        sel_dst_od.reshape(ESEL, 1), sel_dst_od.reshape(1, ESEL),
        odNum, lt_W, mlp_wb)
    return predFlow, selectProb
